# baseline (speedup 1.0000x reference)
import functools

import jax
import jax.numpy as jnp
from jax import lax
from jax.experimental import pallas as pl
from jax.experimental.pallas import tpu as pltpu
from jax.experimental.pallas import tpu_sc as plsc

BATCH = 16384
EMB = 32
NC = 2
NS = 16
NW = NC * NS
BPW = BATCH // NW
V = 1000001
WB = 8192
NBLK = 31
GP = NBLK * WB
VQ = 4 * GP
LAST_BLK = (V - 1) // WB


def _repack_body(x0, x1, x2, x3, out_ref):
    X = jnp.concatenate([x0[...], x1[...], x2[...], x3[...]], axis=0)
    out_ref[...] = X.T.astype(jnp.bfloat16)


def _repack(tT):
    mk = lambda a: pl.BlockSpec(
        (32, WB), lambda i, a=a: (0, jnp.minimum(a * NBLK + i, LAST_BLK)))
    return pl.pallas_call(
        _repack_body,
        grid=(NBLK,),
        in_specs=[mk(0), mk(1), mk(2), mk(3)],
        out_specs=pl.BlockSpec((WB, 128), lambda i: (i, 0)),
        out_shape=jax.ShapeDtypeStruct((GP, 128), jnp.bfloat16),
    )(tT, tT, tT, tT)


def _gather_body(ru_hbm, rm_hbm, qu_hbm, qm_hbm, pu_hbm, pm_hbm,
                 idx_v, rows_v, sem):
    wid = lax.axis_index("s") * NC + lax.axis_index("c")
    base = wid * BPW
    pltpu.sync_copy(qu_hbm.at[pl.ds(base, BPW)], idx_v)
    pltpu.async_copy(ru_hbm.at[idx_v], rows_v, sem).wait()
    pltpu.sync_copy(rows_v, pu_hbm.at[pl.ds(base, BPW)])
    pltpu.sync_copy(qm_hbm.at[pl.ds(base, BPW)], idx_v)
    pltpu.async_copy(rm_hbm.at[idx_v], rows_v, sem).wait()
    pltpu.sync_copy(rows_v, pm_hbm.at[pl.ds(base, BPW)])


_sc_gather = pl.kernel(
    _gather_body,
    out_type=(
        jax.ShapeDtypeStruct((BATCH, EMB), jnp.bfloat16),
        jax.ShapeDtypeStruct((BATCH, EMB), jnp.bfloat16),
    ),
    mesh=plsc.VectorSubcoreMesh(core_axis_name="c", subcore_axis_name="s"),
    compiler_params=pltpu.CompilerParams(use_tc_tiling_on_sc=False),
    scratch_types=[
        pltpu.VMEM((BPW,), jnp.int32),
        pltpu.VMEM((BPW, EMB), jnp.bfloat16),
        pltpu.SemaphoreType.DMA,
    ],
)


def _mlp_body(ue_ref, me_ref, w1u_ref, w1m_ref, b1_ref, w2_ref, b2_ref,
              w3_ref, b3_ref, out_ref):
    ue = ue_ref[...].astype(jnp.float32)
    me = me_ref[...].astype(jnp.float32)
    h1 = jnp.dot(ue, w1u_ref[...], preferred_element_type=jnp.float32)
    h1 += jnp.dot(me, w1m_ref[...], preferred_element_type=jnp.float32)
    h1 = jnp.maximum(h1 + b1_ref[...], 0.0)
    h2 = jnp.dot(h1, w2_ref[...], preferred_element_type=jnp.float32)
    h2 = jnp.maximum(h2 + b2_ref[...], 0.0)
    out_ref[...] = (
        jnp.dot(h2, w3_ref[...], preferred_element_type=jnp.float32)
        + b3_ref[...]
    )


def _mlp(ue, me, w1u, w1m, b1, w2, b2, w3, b3, block=2048):
    grid = BATCH // block
    full = lambda shape: pl.BlockSpec(shape, lambda i: (0, 0))
    return pl.pallas_call(
        _mlp_body,
        grid=(grid,),
        in_specs=[
            pl.BlockSpec((block, EMB), lambda i: (i, 0)),
            pl.BlockSpec((block, EMB), lambda i: (i, 0)),
            full((EMB, 128)),
            full((EMB, 128)),
            full((1, 128)),
            full((128, 64)),
            full((1, 64)),
            full((64, 1)),
            full((1, 1)),
        ],
        out_specs=pl.BlockSpec((block, 1), lambda i: (i, 0)),
        out_shape=jax.ShapeDtypeStruct((BATCH, 1), jnp.float32),
    )(ue, me, w1u, w1m, b1, w2, b2, w3, b3)


def _q_of(r):
    r = r.astype(jnp.int32)
    a = ((r >= GP).astype(jnp.int32) + (r >= 2 * GP).astype(jnp.int32)
         + (r >= 3 * GP).astype(jnp.int32))
    return 4 * (r - a * GP) + a


def kernel(user, movie, user_table, movie_table, W1, b1, W2, b2, W3, b3):
    ru = _repack(user_table.T).reshape(VQ, EMB)
    rm = _repack(movie_table.T).reshape(VQ, EMB)
    ue, me = _sc_gather(ru, rm, _q_of(user), _q_of(movie))
    w1u = W1[:, :EMB].T
    w1m = W1[:, EMB:].T
    out = _mlp(ue, me, w1u, w1m, b1[None, :], W2.T, b2[None, :],
               W3.T, b3[None, :])
    return out[:, 0]


# speedup vs baseline: 2.4009x; 2.4009x over previous
import functools

import jax
import jax.numpy as jnp
from jax import lax
from jax.experimental import pallas as pl
from jax.experimental.pallas import tpu as pltpu
from jax.experimental.pallas import tpu_sc as plsc

BATCH = 16384
EMB = 32
NC = 2
NS = 16
NW = NC * NS
BPW = BATCH // NW
V = 1000001
WB = 4096
NBLK = 62
GP = NBLK * WB
VQ = 4 * GP
LAST_BLK = (V - 1) // WB


def _repack_body(x0, x1, x2, x3, out_ref):
    X = jnp.concatenate([x0[...], x1[...], x2[...], x3[...]], axis=0)
    out_ref[...] = X.T


def _repack(tT):
    mk = lambda a: pl.BlockSpec(
        (32, WB), lambda i, a=a: (0, jnp.minimum(a * NBLK + i, LAST_BLK)))
    return pl.pallas_call(
        _repack_body,
        grid=(NBLK,),
        in_specs=[mk(0), mk(1), mk(2), mk(3)],
        out_specs=pl.BlockSpec((WB, 128), lambda i: (i, 0)),
        out_shape=jax.ShapeDtypeStruct((GP, 128), jnp.float32),
    )(tT, tT, tT, tT)


def _gather_body(ru_hbm, rm_hbm, qu_hbm, qm_hbm, pu_hbm, pm_hbm,
                 idx_v, rows_v, sem):
    wid = lax.axis_index("s") * NC + lax.axis_index("c")
    base = wid * BPW
    pltpu.sync_copy(qu_hbm.at[pl.ds(base, BPW)], idx_v)
    pltpu.async_copy(ru_hbm.at[idx_v], rows_v, sem).wait()
    pltpu.sync_copy(rows_v, pu_hbm.at[pl.ds(base, BPW)])
    pltpu.sync_copy(qm_hbm.at[pl.ds(base, BPW)], idx_v)
    pltpu.async_copy(rm_hbm.at[idx_v], rows_v, sem).wait()
    pltpu.sync_copy(rows_v, pm_hbm.at[pl.ds(base, BPW)])


_sc_gather = pl.kernel(
    _gather_body,
    out_type=(
        jax.ShapeDtypeStruct((BATCH, EMB), jnp.float32),
        jax.ShapeDtypeStruct((BATCH, EMB), jnp.float32),
    ),
    mesh=plsc.VectorSubcoreMesh(core_axis_name="c", subcore_axis_name="s"),
    compiler_params=pltpu.CompilerParams(use_tc_tiling_on_sc=False),
    scratch_types=[
        pltpu.VMEM((BPW,), jnp.int32),
        pltpu.VMEM((BPW, EMB), jnp.float32),
        pltpu.SemaphoreType.DMA,
    ],
)


def _mlp_body(ue_ref, me_ref, w1u_ref, w1m_ref, b1_ref, w2_ref, b2_ref,
              w3_ref, b3_ref, out_ref):
    h1 = jnp.dot(ue_ref[...], w1u_ref[...], preferred_element_type=jnp.float32)
    h1 += jnp.dot(me_ref[...], w1m_ref[...], preferred_element_type=jnp.float32)
    h1 = jnp.maximum(h1 + b1_ref[...], 0.0)
    h2 = jnp.dot(h1, w2_ref[...], preferred_element_type=jnp.float32)
    h2 = jnp.maximum(h2 + b2_ref[...], 0.0)
    out_ref[...] = (
        jnp.dot(h2, w3_ref[...], preferred_element_type=jnp.float32)
        + b3_ref[...]
    )


def _mlp(ue, me, w1u, w1m, b1, w2, b2, w3, b3, block=2048):
    grid = BATCH // block
    full = lambda shape: pl.BlockSpec(shape, lambda i: (0, 0))
    return pl.pallas_call(
        _mlp_body,
        grid=(grid,),
        in_specs=[
            pl.BlockSpec((block, EMB), lambda i: (i, 0)),
            pl.BlockSpec((block, EMB), lambda i: (i, 0)),
            full((EMB, 128)),
            full((EMB, 128)),
            full((1, 128)),
            full((128, 64)),
            full((1, 64)),
            full((64, 1)),
            full((1, 1)),
        ],
        out_specs=pl.BlockSpec((block, 1), lambda i: (i, 0)),
        out_shape=jax.ShapeDtypeStruct((BATCH, 1), jnp.float32),
    )(ue, me, w1u, w1m, b1, w2, b2, w3, b3)


def _q_of(r):
    r = r.astype(jnp.int32)
    a = ((r >= GP).astype(jnp.int32) + (r >= 2 * GP).astype(jnp.int32)
         + (r >= 3 * GP).astype(jnp.int32))
    return 4 * (r - a * GP) + a


def kernel(user, movie, user_table, movie_table, W1, b1, W2, b2, W3, b3):
    ru = _repack(user_table.T).reshape(VQ, EMB)
    rm = _repack(movie_table.T).reshape(VQ, EMB)
    ue, me = _sc_gather(ru, rm, _q_of(user), _q_of(movie))
    w1u = W1[:, :EMB].T
    w1m = W1[:, EMB:].T
    out = _mlp(ue, me, w1u, w1m, b1[None, :], W2.T, b2[None, :],
               W3.T, b3[None, :])
    return out[:, 0]


# speedup vs baseline: 2.7346x; 1.1390x over previous
import functools

import jax
import jax.numpy as jnp
from jax import lax
from jax.experimental import pallas as pl
from jax.experimental.pallas import tpu as pltpu
from jax.experimental.pallas import tpu_sc as plsc

BATCH = 16384
EMB = 32
NC = 2
NS = 16
NW = NC * NS
BPW = BATCH // NW
V = 1000001
WB = 16384
NBLK = 16
GP = NBLK * WB
VQ = 4 * GP
LAST_BLK = (V - 1) // WB


def _repack_body(x0, x1, x2, x3, out_ref):
    X = jnp.concatenate([x0[...], x1[...], x2[...], x3[...]], axis=0)
    out_ref[...] = X.T


def _repack(tT):
    mk = lambda a: pl.BlockSpec(
        (32, WB), lambda i, a=a: (0, jnp.minimum(a * NBLK + i, LAST_BLK)))
    return pl.pallas_call(
        _repack_body,
        grid=(NBLK,),
        in_specs=[mk(0), mk(1), mk(2), mk(3)],
        out_specs=pl.BlockSpec((WB, 128), lambda i: (i, 0)),
        out_shape=jax.ShapeDtypeStruct((GP, 128), jnp.float32),
    )(tT, tT, tT, tT)


def _gather_body(ru_hbm, rm_hbm, qu_hbm, qm_hbm, pu_hbm, pm_hbm,
                 idx_v, rows_v, sem):
    wid = lax.axis_index("s") * NC + lax.axis_index("c")
    base = wid * BPW
    pltpu.sync_copy(qu_hbm.at[pl.ds(base, BPW)], idx_v)
    pltpu.async_copy(ru_hbm.at[idx_v], rows_v, sem).wait()
    pltpu.sync_copy(rows_v, pu_hbm.at[pl.ds(base, BPW)])
    pltpu.sync_copy(qm_hbm.at[pl.ds(base, BPW)], idx_v)
    pltpu.async_copy(rm_hbm.at[idx_v], rows_v, sem).wait()
    pltpu.sync_copy(rows_v, pm_hbm.at[pl.ds(base, BPW)])


_sc_gather = pl.kernel(
    _gather_body,
    out_type=(
        jax.ShapeDtypeStruct((BATCH, EMB), jnp.float32),
        jax.ShapeDtypeStruct((BATCH, EMB), jnp.float32),
    ),
    mesh=plsc.VectorSubcoreMesh(core_axis_name="c", subcore_axis_name="s"),
    compiler_params=pltpu.CompilerParams(use_tc_tiling_on_sc=False),
    scratch_types=[
        pltpu.VMEM((BPW,), jnp.int32),
        pltpu.VMEM((BPW, EMB), jnp.float32),
        pltpu.SemaphoreType.DMA,
    ],
)


def _mlp_body(ue_ref, me_ref, w1u_ref, w1m_ref, b1_ref, w2_ref, b2_ref,
              w3_ref, b3_ref, out_ref):
    h1 = jnp.dot(ue_ref[...], w1u_ref[...], preferred_element_type=jnp.float32)
    h1 += jnp.dot(me_ref[...], w1m_ref[...], preferred_element_type=jnp.float32)
    h1 = jnp.maximum(h1 + b1_ref[...], 0.0)
    h2 = jnp.dot(h1, w2_ref[...], preferred_element_type=jnp.float32)
    h2 = jnp.maximum(h2 + b2_ref[...], 0.0)
    out_ref[...] = (
        jnp.dot(h2, w3_ref[...], preferred_element_type=jnp.float32)
        + b3_ref[...]
    )


def _mlp(ue, me, w1u, w1m, b1, w2, b2, w3, b3, block=2048):
    grid = BATCH // block
    full = lambda shape: pl.BlockSpec(shape, lambda i: (0, 0))
    return pl.pallas_call(
        _mlp_body,
        grid=(grid,),
        in_specs=[
            pl.BlockSpec((block, EMB), lambda i: (i, 0)),
            pl.BlockSpec((block, EMB), lambda i: (i, 0)),
            full((EMB, 128)),
            full((EMB, 128)),
            full((1, 128)),
            full((128, 64)),
            full((1, 64)),
            full((64, 1)),
            full((1, 1)),
        ],
        out_specs=pl.BlockSpec((block, 1), lambda i: (i, 0)),
        out_shape=jax.ShapeDtypeStruct((BATCH, 1), jnp.float32),
    )(ue, me, w1u, w1m, b1, w2, b2, w3, b3)


def _q_of(r):
    r = r.astype(jnp.int32)
    a = ((r >= GP).astype(jnp.int32) + (r >= 2 * GP).astype(jnp.int32)
         + (r >= 3 * GP).astype(jnp.int32))
    return 4 * (r - a * GP) + a


def kernel(user, movie, user_table, movie_table, W1, b1, W2, b2, W3, b3):
    ru = _repack(user_table.T).reshape(VQ, EMB)
    rm = _repack(movie_table.T).reshape(VQ, EMB)
    ue, me = _sc_gather(ru, rm, _q_of(user), _q_of(movie))
    w1u = W1[:, :EMB].T
    w1m = W1[:, EMB:].T
    out = _mlp(ue, me, w1u, w1m, b1[None, :], W2.T, b2[None, :],
               W3.T, b3[None, :])
    return out[:, 0]


# speedup vs baseline: 2.7391x; 1.0017x over previous
import functools

import jax
import jax.numpy as jnp
from jax import lax
from jax.experimental import pallas as pl
from jax.experimental.pallas import tpu as pltpu
from jax.experimental.pallas import tpu_sc as plsc

BATCH = 16384
EMB = 32
NC = 2
NS = 16
NW = NC * NS
BPW = BATCH // NW
V = 1000001
WB = 16384
NBLK = 16
GP = NBLK * WB
VQ = 4 * GP
LAST_BLK = (V - 1) // WB


def _repack_body(x0, x1, x2, x3, out_ref):
    X = jnp.concatenate([x0[...], x1[...], x2[...], x3[...]], axis=0)
    out_ref[...] = X.T


def _repack(tT):
    mk = lambda a: pl.BlockSpec(
        (32, WB), lambda i, a=a: (0, jnp.minimum(a * NBLK + i, LAST_BLK)))
    return pl.pallas_call(
        _repack_body,
        grid=(NBLK,),
        in_specs=[mk(0), mk(1), mk(2), mk(3)],
        out_specs=pl.BlockSpec((WB, 128), lambda i: (i, 0)),
        out_shape=jax.ShapeDtypeStruct((GP, 128), jnp.float32),
    )(tT, tT, tT, tT)


def _gather_body(r_hbm, q_hbm, p_hbm, idx_v, rows_v, sem):
    wid = lax.axis_index("s") * NC + lax.axis_index("c")
    base = wid * BPW
    pltpu.sync_copy(q_hbm.at[pl.ds(base, BPW)], idx_v)
    pltpu.async_copy(r_hbm.at[idx_v], rows_v, sem).wait()
    pltpu.sync_copy(rows_v, p_hbm.at[pl.ds(base, BPW)])


_sc_gather1 = pl.kernel(
    _gather_body,
    out_type=jax.ShapeDtypeStruct((BATCH, EMB), jnp.float32),
    mesh=plsc.VectorSubcoreMesh(core_axis_name="c", subcore_axis_name="s"),
    compiler_params=pltpu.CompilerParams(use_tc_tiling_on_sc=False),
    scratch_types=[
        pltpu.VMEM((BPW,), jnp.int32),
        pltpu.VMEM((BPW, EMB), jnp.float32),
        pltpu.SemaphoreType.DMA,
    ],
)


def _mlp_body(ue_ref, me_ref, w1u_ref, w1m_ref, b1_ref, w2_ref, b2_ref,
              w3_ref, b3_ref, out_ref):
    h1 = jnp.dot(ue_ref[...], w1u_ref[...], preferred_element_type=jnp.float32)
    h1 += jnp.dot(me_ref[...], w1m_ref[...], preferred_element_type=jnp.float32)
    h1 = jnp.maximum(h1 + b1_ref[...], 0.0)
    h2 = jnp.dot(h1, w2_ref[...], preferred_element_type=jnp.float32)
    h2 = jnp.maximum(h2 + b2_ref[...], 0.0)
    out_ref[...] = (
        jnp.dot(h2, w3_ref[...], preferred_element_type=jnp.float32)
        + b3_ref[...]
    )


def _mlp(ue, me, w1u, w1m, b1, w2, b2, w3, b3, block=2048):
    grid = BATCH // block
    full = lambda shape: pl.BlockSpec(shape, lambda i: (0, 0))
    return pl.pallas_call(
        _mlp_body,
        grid=(grid,),
        in_specs=[
            pl.BlockSpec((block, EMB), lambda i: (i, 0)),
            pl.BlockSpec((block, EMB), lambda i: (i, 0)),
            full((EMB, 128)),
            full((EMB, 128)),
            full((1, 128)),
            full((128, 64)),
            full((1, 64)),
            full((64, 1)),
            full((1, 1)),
        ],
        out_specs=pl.BlockSpec((block, 1), lambda i: (i, 0)),
        out_shape=jax.ShapeDtypeStruct((BATCH, 1), jnp.float32),
    )(ue, me, w1u, w1m, b1, w2, b2, w3, b3)


def _q_of(r):
    r = r.astype(jnp.int32)
    a = ((r >= GP).astype(jnp.int32) + (r >= 2 * GP).astype(jnp.int32)
         + (r >= 3 * GP).astype(jnp.int32))
    return 4 * (r - a * GP) + a


def kernel(user, movie, user_table, movie_table, W1, b1, W2, b2, W3, b3):
    ru = _repack(user_table.T).reshape(VQ, EMB)
    ue = _sc_gather1(ru, _q_of(user))
    rm = _repack(movie_table.T).reshape(VQ, EMB)
    me = _sc_gather1(rm, _q_of(movie))
    w1u = W1[:, :EMB].T
    w1m = W1[:, EMB:].T
    out = _mlp(ue, me, w1u, w1m, b1[None, :], W2.T, b2[None, :],
               W3.T, b3[None, :])
    return out[:, 0]


# speedup vs baseline: 2.7396x; 1.0002x over previous
import jax
import jax.numpy as jnp
from jax import lax
from jax.experimental import pallas as pl
from jax.experimental.pallas import tpu as pltpu
from jax.experimental.pallas import tpu_sc as plsc

BATCH = 16384
EMB = 32
NC = 2
NS = 16
NW = NC * NS
BPW = BATCH // NW
V = 1000001
WB = 16384
NBLK = 16
GP = NBLK * WB
VQ = 4 * GP
LAST_BLK = (V - 1) // WB


def _repack_body(x0, x1, x2, x3, out_ref):
    X = jnp.concatenate([x0[...], x1[...], x2[...], x3[...]], axis=0)
    out_ref[...] = X.T


def _repack(tT):
    mk = lambda a: pl.BlockSpec(
        (32, WB), lambda i, a=a: (0, jnp.minimum(a * NBLK + i, LAST_BLK)))
    return pl.pallas_call(
        _repack_body,
        grid=(NBLK,),
        in_specs=[mk(0), mk(1), mk(2), mk(3)],
        out_specs=pl.BlockSpec((WB, 128), lambda i: (i, 0)),
        out_shape=jax.ShapeDtypeStruct((GP, 128), jnp.float32),
    )(tT, tT, tT, tT)


def _gather_body(r_hbm, q_hbm, p_hbm, idx_v, rows_v, sem):
    wid = lax.axis_index("s") * NC + lax.axis_index("c")
    base = wid * BPW
    pltpu.sync_copy(q_hbm.at[pl.ds(base, BPW)], idx_v)
    pltpu.async_copy(r_hbm.at[idx_v], rows_v, sem).wait()
    pltpu.sync_copy(rows_v, p_hbm.at[pl.ds(base, BPW)])


_sc_gather1 = pl.kernel(
    _gather_body,
    out_type=jax.ShapeDtypeStruct((BATCH, EMB), jnp.float32),
    mesh=plsc.VectorSubcoreMesh(core_axis_name="c", subcore_axis_name="s"),
    compiler_params=pltpu.CompilerParams(use_tc_tiling_on_sc=False),
    scratch_types=[
        pltpu.VMEM((BPW,), jnp.int32),
        pltpu.VMEM((BPW, EMB), jnp.float32),
        pltpu.SemaphoreType.DMA,
    ],
)


def _mlp_body(ue_ref, me_ref, w1u_ref, w1m_ref, b1_ref, w2_ref, b2_ref,
              w3_ref, b3_ref, out_ref):
    h1 = jnp.dot(ue_ref[...], w1u_ref[...], preferred_element_type=jnp.float32)
    h1 += jnp.dot(me_ref[...], w1m_ref[...], preferred_element_type=jnp.float32)
    h1 = jnp.maximum(h1 + b1_ref[...], 0.0)
    h2 = jnp.dot(h1, w2_ref[...], preferred_element_type=jnp.float32)
    h2 = jnp.maximum(h2 + b2_ref[...], 0.0)
    out_ref[...] = (
        jnp.dot(h2, w3_ref[...], preferred_element_type=jnp.float32)
        + b3_ref[...]
    )


def _mlp(ue, me, w1u, w1m, b1, w2, b2, w3, b3, block=2048):
    grid = BATCH // block
    full = lambda shape: pl.BlockSpec(shape, lambda i: (0, 0))
    return pl.pallas_call(
        _mlp_body,
        grid=(grid,),
        in_specs=[
            pl.BlockSpec((block, EMB), lambda i: (i, 0)),
            pl.BlockSpec((block, EMB), lambda i: (i, 0)),
            full((EMB, 128)),
            full((EMB, 128)),
            full((1, 128)),
            full((128, 64)),
            full((1, 64)),
            full((64, 1)),
            full((1, 1)),
        ],
        out_specs=pl.BlockSpec((block, 1), lambda i: (i, 0)),
        out_shape=jax.ShapeDtypeStruct((BATCH, 1), jnp.float32),
    )(ue, me, w1u, w1m, b1, w2, b2, w3, b3)


def _q_of(r):
    r = r.astype(jnp.int32)
    a = ((r >= GP).astype(jnp.int32) + (r >= 2 * GP).astype(jnp.int32)
         + (r >= 3 * GP).astype(jnp.int32))
    return 4 * (r - a * GP) + a


def kernel(user, movie, user_table, movie_table, W1, b1, W2, b2, W3, b3):
    ru = _repack(user_table.T).reshape(VQ, EMB)
    ue = _sc_gather1(ru, _q_of(user))
    rm = _repack(movie_table.T).reshape(VQ, EMB)
    me = _sc_gather1(rm, _q_of(movie))
    w1u = W1[:, :EMB].T
    w1m = W1[:, EMB:].T
    out = _mlp(ue, me, w1u, w1m, b1[None, :], W2.T, b2[None, :],
               W3.T, b3[None, :])
    return out[:, 0]
